# baseline (speedup 1.0000x reference)
import functools

import jax
import jax.numpy as jnp
from jax import lax
from jax.experimental import pallas as pl
from jax.experimental.pallas import tpu as pltpu
from jax.experimental.pallas import tpu_sc as plsc

N = 32768
D_NODE = 128
D_HID = 1024
K = 64


def _rank_body(nodes_ref, hid_ref, w_ref, b_ref, sel_ref):
    nodes64 = nodes_ref[...]
    hid = hid_ref[...]
    W = w_ref[...]
    b = b_ref[...]

    f32 = jnp.float32
    h = lax.dot_general(hid, W, (((1,), (1,)), ((), ())),
                        preferred_element_type=f32) + b
    s_row = lax.dot_general(h, nodes64, (((1,), (1,)), ((), ())),
                            preferred_element_type=f32)

    I = lax.broadcasted_iota(jnp.int32, (K, K), 0)
    J = lax.broadcasted_iota(jnp.int32, (K, K), 1)

    s_col = jnp.transpose(s_row, (1, 0))
    S1 = jnp.broadcast_to(s_col, (K, K))
    S2 = jnp.broadcast_to(s_row, (K, K))

    C = ((S1 < S2) | ((S1 == S2) & (I < J))).astype(f32)
    rank_row = jnp.sum(C, axis=0, keepdims=True)
    rank_mat = jnp.broadcast_to(rank_row, (K, K)).astype(jnp.int32)
    P = (rank_mat == I).astype(f32)
    ival = lax.broadcasted_iota(jnp.int32, (1, K), 1).astype(f32)
    sel = lax.dot_general(ival, P, (((1,), (1,)), ((), ())),
                          preferred_element_type=f32,
                          precision=lax.Precision.HIGHEST)
    sel_ref[...] = sel.astype(jnp.int32)


_sc_mesh = plsc.VectorSubcoreMesh(core_axis_name="c", subcore_axis_name="s")


@functools.partial(
    pl.kernel,
    mesh=_sc_mesh,
    out_type=jax.ShapeDtypeStruct((K, D_NODE), jnp.float32),
    scratch_types=[
        pltpu.VMEM((K,), jnp.int32),
        pltpu.VMEM((K, D_NODE), jnp.float32),
        pltpu.SemaphoreType.DMA,
    ],
)
def _sc_gather(nodes_hbm, idx_hbm, out_hbm, idx_v, rows_v, sem):
    wid = lax.axis_index("s") * 2 + lax.axis_index("c")

    @pl.when(wid == 0)
    def _():
        pltpu.sync_copy(idx_hbm, idx_v)
        pltpu.async_copy(nodes_hbm.at[idx_v], rows_v, sem).wait()
        pltpu.sync_copy(rows_v, out_hbm)


def kernel(nodes, hidden_state, W, b):
    sel = pl.pallas_call(
        _rank_body,
        grid=(1,),
        in_specs=[
            pl.BlockSpec((K, D_NODE), lambda i: (0, 0)),
            pl.BlockSpec((1, D_HID), lambda i: (0, 0)),
            pl.BlockSpec((D_NODE, D_HID), lambda i: (0, 0)),
            pl.BlockSpec((1, D_NODE), lambda i: (0, 0)),
        ],
        out_specs=pl.BlockSpec((1, K), lambda i: (0, 0)),
        out_shape=jax.ShapeDtypeStruct((1, K), jnp.int32),
    )(nodes, hidden_state, W, b.reshape(1, D_NODE))
    out = _sc_gather(nodes, sel.reshape(K))
    return out.reshape(1, K * D_NODE)


# speedup vs baseline: 1.0709x; 1.0709x over previous
import functools

import jax
import jax.numpy as jnp
from jax import lax
from jax.experimental import pallas as pl
from jax.experimental.pallas import tpu as pltpu
from jax.experimental.pallas import tpu_sc as plsc

N = 32768
D_NODE = 128
D_HID = 1024
K = 64


def _rank_body(nodes_ref, hid_ref, w_ref, b_ref, sel_ref):
    nodes64 = nodes_ref[...]
    hid = hid_ref[...]
    W = w_ref[...]
    b = b_ref[...]

    f32 = jnp.float32
    h = lax.dot_general(hid, W, (((1,), (1,)), ((), ())),
                        preferred_element_type=f32) + b
    s_row = lax.dot_general(h, nodes64, (((1,), (1,)), ((), ())),
                            preferred_element_type=f32)

    I = lax.broadcasted_iota(jnp.int32, (K, K), 0)
    J = lax.broadcasted_iota(jnp.int32, (K, K), 1)

    s_col = jnp.transpose(s_row, (1, 0))
    S1 = jnp.broadcast_to(s_col, (K, K))
    S2 = jnp.broadcast_to(s_row, (K, K))

    C = ((S1 < S2) | ((S1 == S2) & (I < J))).astype(f32)
    rank_row = jnp.sum(C, axis=0, keepdims=True)
    rank_mat = jnp.broadcast_to(rank_row, (K, K)).astype(jnp.int32)
    P = (rank_mat == I).astype(f32)
    ival = lax.broadcasted_iota(jnp.int32, (1, K), 1).astype(f32)
    sel = lax.dot_general(ival, P, (((1,), (1,)), ((), ())),
                          preferred_element_type=f32,
                          precision=lax.Precision.HIGHEST)
    sel_ref[...] = sel.astype(jnp.int32)


_sc_mesh = plsc.VectorSubcoreMesh(core_axis_name="c", subcore_axis_name="s",
                                  num_cores=1)


@functools.partial(
    pl.kernel,
    mesh=_sc_mesh,
    out_type=jax.ShapeDtypeStruct((K, D_NODE), jnp.float32),
    scratch_types=[
        pltpu.VMEM((K,), jnp.int32),
        pltpu.VMEM((K, D_NODE), jnp.float32),
        pltpu.SemaphoreType.DMA,
    ],
)
def _sc_gather(nodes_hbm, idx_hbm, out_hbm, idx_v, rows_v, sem):
    wid = lax.axis_index("s")

    @pl.when(wid == 0)
    def _():
        pltpu.sync_copy(idx_hbm.at[0], idx_v)
        pltpu.async_copy(nodes_hbm.at[idx_v], rows_v, sem).wait()
        pltpu.sync_copy(rows_v, out_hbm)


def kernel(nodes, hidden_state, W, b):
    sel = pl.pallas_call(
        _rank_body,
        grid=(1,),
        in_specs=[
            pl.BlockSpec((K, D_NODE), lambda i: (0, 0)),
            pl.BlockSpec((1, D_HID), lambda i: (0, 0)),
            pl.BlockSpec((D_NODE, D_HID), lambda i: (0, 0)),
            pl.BlockSpec((1, D_NODE), lambda i: (0, 0)),
        ],
        out_specs=pl.BlockSpec((1, K), lambda i: (0, 0)),
        out_shape=jax.ShapeDtypeStruct((1, K), jnp.int32),
    )(nodes, hidden_state, W, b.reshape(1, D_NODE))
    out = _sc_gather(nodes, sel)
    return out.reshape(1, K * D_NODE)


# speedup vs baseline: 1.1013x; 1.0284x over previous
import functools

import jax
import jax.numpy as jnp
from jax import lax
from jax.experimental import pallas as pl
from jax.experimental.pallas import tpu as pltpu
from jax.experimental.pallas import tpu_sc as plsc

N = 32768
D_NODE = 128
D_HID = 1024
K = 64


def _rank_body(nodes_ref, hid_ref, w_ref, b_ref, sel_ref):
    nodes64 = nodes_ref[...]
    hid = hid_ref[...]
    W = w_ref[...]
    b = b_ref[...]

    f32 = jnp.float32
    h = lax.dot_general(hid, W, (((1,), (1,)), ((), ())),
                        preferred_element_type=f32) + b
    s_row = lax.dot_general(h, nodes64, (((1,), (1,)), ((), ())),
                            preferred_element_type=f32)

    I = lax.broadcasted_iota(jnp.int32, (K, K), 0)
    J = lax.broadcasted_iota(jnp.int32, (K, K), 1)

    s_col = jnp.transpose(s_row, (1, 0))
    S1 = jnp.broadcast_to(s_col, (K, K))
    S2 = jnp.broadcast_to(s_row, (K, K))

    C = ((S1 < S2) | ((S1 == S2) & (I < J))).astype(f32)
    rank_row = jnp.sum(C, axis=0, keepdims=True)
    rank_mat = jnp.broadcast_to(rank_row, (K, K)).astype(jnp.int32)
    P = (rank_mat == I).astype(f32)
    ival = lax.broadcasted_iota(jnp.int32, (1, K), 1).astype(f32)
    sel = lax.dot_general(ival, P, (((1,), (1,)), ((), ())),
                          preferred_element_type=f32,
                          precision=lax.Precision.HIGHEST)
    sel_ref[...] = sel.astype(jnp.int32)


_sc_mesh = plsc.VectorSubcoreMesh(core_axis_name="c", subcore_axis_name="s",
                                  num_cores=1)


@functools.partial(
    pl.kernel,
    mesh=_sc_mesh,
    out_type=jax.ShapeDtypeStruct((K, D_NODE), jnp.float32),
    scratch_types=[
        pltpu.VMEM((8,), jnp.int32),
        pltpu.VMEM((8, D_NODE), jnp.float32),
        pltpu.SemaphoreType.DMA,
    ],
)
def _sc_gather(nodes_hbm, idx_hbm, out_hbm, idx_v, rows_v, sem):
    wid = lax.axis_index("s")

    @pl.when(wid < 8)
    def _():
        base = wid * 8
        pltpu.sync_copy(idx_hbm.at[0, pl.ds(base, 8)], idx_v)
        pltpu.async_copy(nodes_hbm.at[idx_v], rows_v, sem).wait()
        pltpu.sync_copy(rows_v, out_hbm.at[pl.ds(base, 8)])


def kernel(nodes, hidden_state, W, b):
    sel = pl.pallas_call(
        _rank_body,
        grid=(1,),
        in_specs=[
            pl.BlockSpec((K, D_NODE), lambda i: (0, 0)),
            pl.BlockSpec((1, D_HID), lambda i: (0, 0)),
            pl.BlockSpec((D_NODE, D_HID), lambda i: (0, 0)),
            pl.BlockSpec((1, D_NODE), lambda i: (0, 0)),
        ],
        out_specs=pl.BlockSpec((1, K), lambda i: (0, 0)),
        out_shape=jax.ShapeDtypeStruct((1, K), jnp.int32),
    )(nodes, hidden_state, W, b.reshape(1, D_NODE))
    out = _sc_gather(nodes, sel)
    return out.reshape(1, K * D_NODE)


# speedup vs baseline: 1.1176x; 1.0148x over previous
import functools

import jax
import jax.numpy as jnp
from jax import lax
from jax.experimental import pallas as pl
from jax.experimental.pallas import tpu as pltpu
from jax.experimental.pallas import tpu_sc as plsc

N = 32768
D_NODE = 128
D_HID = 1024
K = 64


def _rank_body(nodes_ref, hid_ref, w_ref, b_ref, sel_ref):
    nodes64 = nodes_ref[...]
    hid = hid_ref[...]
    W = w_ref[...]
    b = b_ref[...]

    f32 = jnp.float32
    h = lax.dot_general(hid, W, (((1,), (1,)), ((), ())),
                        preferred_element_type=f32) + b
    s_row = lax.dot_general(h, nodes64, (((1,), (1,)), ((), ())),
                            preferred_element_type=f32)

    I = lax.broadcasted_iota(jnp.int32, (K, K), 0)
    J = lax.broadcasted_iota(jnp.int32, (K, K), 1)

    s_col = jnp.transpose(s_row, (1, 0))
    S1 = jnp.broadcast_to(s_col, (K, K))
    S2 = jnp.broadcast_to(s_row, (K, K))

    C = ((S1 < S2) | ((S1 == S2) & (I < J))).astype(f32)
    rank_row = jnp.sum(C, axis=0, keepdims=True)
    rank_mat = jnp.broadcast_to(rank_row, (K, K)).astype(jnp.int32)
    P = (rank_mat == I).astype(f32)
    ival = lax.broadcasted_iota(jnp.int32, (1, K), 1).astype(f32)
    sel = lax.dot_general(ival, P, (((1,), (1,)), ((), ())),
                          preferred_element_type=f32,
                          precision=lax.Precision.HIGHEST)
    sel_ref[...] = sel.astype(jnp.int32)


_sc_mesh = plsc.ScalarSubcoreMesh(axis_name="c", num_cores=1)


@functools.partial(
    pl.kernel,
    mesh=_sc_mesh,
    out_type=jax.ShapeDtypeStruct((K, D_NODE), jnp.float32),
    scratch_types=[
        pltpu.SMEM((K,), jnp.int32),
        pltpu.SemaphoreType.DMA,
    ],
)
def _sc_gather(nodes_hbm, idx_hbm, out_hbm, idx_s, sem):
    pltpu.sync_copy(idx_hbm.at[0], idx_s)
    copies = []
    for i in range(K):
        idx = idx_s[i]
        copies.append(pltpu.async_copy(nodes_hbm.at[idx], out_hbm.at[i], sem))
    for c in copies:
        c.wait()


def kernel(nodes, hidden_state, W, b):
    sel = pl.pallas_call(
        _rank_body,
        grid=(1,),
        in_specs=[
            pl.BlockSpec((K, D_NODE), lambda i: (0, 0)),
            pl.BlockSpec((1, D_HID), lambda i: (0, 0)),
            pl.BlockSpec((D_NODE, D_HID), lambda i: (0, 0)),
            pl.BlockSpec((1, D_NODE), lambda i: (0, 0)),
        ],
        out_specs=pl.BlockSpec((1, K), lambda i: (0, 0)),
        out_shape=jax.ShapeDtypeStruct((1, K), jnp.int32),
    )(nodes, hidden_state, W, b.reshape(1, D_NODE))
    out = _sc_gather(nodes, sel)
    return out.reshape(1, K * D_NODE)


# speedup vs baseline: 1.1183x; 1.0007x over previous
import functools

import jax
import jax.numpy as jnp
from jax import lax
from jax.experimental import pallas as pl
from jax.experimental.pallas import tpu as pltpu
from jax.experimental.pallas import tpu_sc as plsc

N = 32768
D_NODE = 128
D_HID = 1024
K = 64


def _rank_body(nodes_ref, hid_ref, w_ref, b_ref, sel_ref):
    nodes64 = nodes_ref[...]
    hid = hid_ref[...]
    W = w_ref[...]
    b = b_ref[...]

    f32 = jnp.float32
    h = lax.dot_general(hid, W, (((1,), (1,)), ((), ())),
                        preferred_element_type=f32) + b
    s_row = lax.dot_general(h, nodes64, (((1,), (1,)), ((), ())),
                            preferred_element_type=f32)

    I = lax.broadcasted_iota(jnp.int32, (K, K), 0)
    J = lax.broadcasted_iota(jnp.int32, (K, K), 1)

    s_col = jnp.transpose(s_row, (1, 0))
    S1 = jnp.broadcast_to(s_col, (K, K))
    S2 = jnp.broadcast_to(s_row, (K, K))

    C = ((S1 < S2) | ((S1 == S2) & (I < J))).astype(f32)
    rank_row = jnp.sum(C, axis=0, keepdims=True)
    rank_mat = jnp.broadcast_to(rank_row, (K, K)).astype(jnp.int32)
    P = (rank_mat == I).astype(f32)
    ival = lax.broadcasted_iota(jnp.int32, (1, K), 1).astype(f32)
    sel = lax.dot_general(ival, P, (((1,), (1,)), ((), ())),
                          preferred_element_type=f32,
                          precision=lax.Precision.HIGHEST)
    sel_ref[...] = sel.astype(jnp.int32)


_sc_mesh = plsc.ScalarSubcoreMesh(axis_name="c", num_cores=1)


@functools.partial(
    pl.kernel,
    mesh=_sc_mesh,
    out_type=jax.ShapeDtypeStruct((K, D_NODE), jnp.float32),
    scratch_types=[
        pltpu.SMEM((K,), jnp.int32),
        pltpu.SemaphoreType.DMA,
    ],
)
def _sc_gather(nodes_hbm, idx_hbm, out_hbm, idx_s, sem):
    pltpu.sync_copy(idx_hbm.at[0], idx_s)

    def issue(i, carry):
        pltpu.async_copy(nodes_hbm.at[idx_s[i]], out_hbm.at[i], sem)
        return carry

    lax.fori_loop(0, K, issue, 0)
    pltpu.make_async_copy(nodes_hbm.at[pl.ds(0, K)], out_hbm, sem).wait()


def kernel(nodes, hidden_state, W, b):
    sel = pl.pallas_call(
        _rank_body,
        grid=(1,),
        in_specs=[
            pl.BlockSpec((K, D_NODE), lambda i: (0, 0)),
            pl.BlockSpec((1, D_HID), lambda i: (0, 0)),
            pl.BlockSpec((D_NODE, D_HID), lambda i: (0, 0)),
            pl.BlockSpec((1, D_NODE), lambda i: (0, 0)),
        ],
        out_specs=pl.BlockSpec((1, K), lambda i: (0, 0)),
        out_shape=jax.ShapeDtypeStruct((1, K), jnp.int32),
    )(nodes, hidden_state, W, b.reshape(1, D_NODE))
    out = _sc_gather(nodes, sel)
    return out.reshape(1, K * D_NODE)
